# Initial kernel scaffold; baseline (speedup 1.0000x reference)
import jax
import jax.numpy as jnp
from jax.experimental import pallas as pl


def kernel(feat, edge_index, feat_mask, edge_mask, W1, W2, pred_value):
    raise NotImplementedError("write your pallas kernel here")


# speedup vs baseline: 3.2751x; 3.2751x over previous
import functools

import jax
import jax.numpy as jnp
from jax import lax
from jax.experimental import pallas as pl
from jax.experimental.pallas import tpu as pltpu
from jax.experimental.pallas import tpu_sc as plsc

N_NODES = 10000
N_EDGES = 160000
D_FEAT = 256
HALF = 128
ALPHA1 = 0.005
ALPHA2 = 1.0
BETA1 = 1.0
BETA2 = 0.1
EPS = 1e-15

NT = 16
E_PER_TILE = N_EDGES // NT
CHUNK = 128
N_FULL = E_PER_TILE // CHUNK
TAIL = E_PER_TILE - N_FULL * CHUNK
ROWS_PER_TILE = 624

@functools.cache
def _make_sc_edge_aggregate():
    mesh = plsc.VectorSubcoreMesh(core_axis_name="c", subcore_axis_name="s")

    @functools.partial(
        pl.kernel,
        mesh=mesh,
        out_type=(
            jax.ShapeDtypeStruct((N_NODES, HALF), jnp.float32),
            jax.ShapeDtypeStruct((N_NODES, HALF), jnp.float32),
        ),
        scratch_types=[
            pltpu.VMEM_SHARED((N_NODES, HALF), jnp.float32),
            pltpu.VMEM((CHUNK,), jnp.int32),
            pltpu.VMEM((CHUNK,), jnp.int32),
            pltpu.VMEM((CHUNK,), jnp.float32),
            pltpu.VMEM((CHUNK, HALF), jnp.float32),
            pltpu.VMEM((16,), jnp.int32),
            pltpu.VMEM((16,), jnp.int32),
            pltpu.VMEM((16,), jnp.float32),
            pltpu.VMEM((16, HALF), jnp.float32),
            pltpu.SemaphoreType.DMA,
        ],
    )
    def _sc_edge_aggregate(feat2, src_h, dst_h, em, out_lo, out_hi,
                           acc, src_v, dst_v, em_v, rows_v,
                           src_t, dst_t, em_t, rows_t, sem):
        _sc_body(feat2, src_h, dst_h, em, out_lo, out_hi,
                 acc, src_v, dst_v, em_v, rows_v,
                 src_t, dst_t, em_t, rows_t, sem)

    return _sc_edge_aggregate


def _sc_body(feat2, src_h, dst_h, em, out_lo, out_hi,
             acc, src_v, dst_v, em_v, rows_v,
             src_t, dst_t, em_t, rows_t, sem):
    c = lax.axis_index("c")
    s = lax.axis_index("s")

    def _zrow(i, carry):
        for j in range(HALF // 16):
            rows_v[i, pl.ds(j * 16, 16)] = jnp.zeros((16,), jnp.float32)
        return carry
    lax.fori_loop(0, CHUNK, _zrow, 0)
    rbase = s * ROWS_PER_TILE
    for t in range(4):
        pltpu.sync_copy(rows_v.at[pl.ds(0, CHUNK)],
                        acc.at[pl.ds(rbase + t * CHUNK, CHUNK)])
    pltpu.sync_copy(rows_v.at[pl.ds(0, ROWS_PER_TILE - 4 * CHUNK)],
                    acc.at[pl.ds(rbase + 4 * CHUNK, ROWS_PER_TILE - 4 * CHUNK)])

    @pl.when(s == NT - 1)
    def _():
        pltpu.sync_copy(rows_v.at[pl.ds(0, N_NODES - NT * ROWS_PER_TILE)],
                        acc.at[pl.ds(NT * ROWS_PER_TILE,
                                     N_NODES - NT * ROWS_PER_TILE)])
    plsc.subcore_barrier()

    def _chunk(base, k, src_r, dst_r, em_r, rows_r):
        pltpu.sync_copy(src_h.at[pl.ds(base, k)], src_r)
        pltpu.sync_copy(dst_h.at[pl.ds(base, k)], dst_r)
        pltpu.sync_copy(em.at[pl.ds(base, k)], em_r)
        for g in range(k // 16):
            sl = pl.ds(g * 16, 16)
            src_r[sl] = src_r[sl] * 2 + c
            x = em_r[sl]
            em_r[sl] = 1.0 / (1.0 + jnp.exp(-x))
        pltpu.async_copy(feat2.at[src_r], rows_r, sem).wait()

        def _scale(g, carry):
            wv = em_r[pl.ds(g * 16, 16)]

            def _lane(l, carry2):
                w = lax.gather(
                    wv, lax.broadcast(l, (16,))[:, None],
                    lax.GatherDimensionNumbers(offset_dims=(),
                                               collapsed_slice_dims=(0,),
                                               start_index_map=(0,)),
                    (1,), mode=lax.GatherScatterMode.PROMISE_IN_BOUNDS)
                e = g * 16 + l
                for j in range(HALF // 16):
                    sl = pl.ds(j * 16, 16)
                    rows_r[e, sl] = rows_r[e, sl] * w
                return carry2
            return lax.fori_loop(0, 16, _lane, carry)
        lax.fori_loop(0, k // 16, _scale, 0)
        pltpu.sync_copy(rows_r, acc.at[dst_r], add=True)

    ebase = s * E_PER_TILE

    def _body(kk, carry):
        _chunk(ebase + kk * CHUNK, CHUNK, src_v, dst_v, em_v, rows_v)
        return carry
    lax.fori_loop(0, N_FULL, _body, 0)
    _chunk(ebase + N_FULL * CHUNK, TAIL, src_t, dst_t, em_t, rows_t)

    plsc.subcore_barrier()
    sl = pl.ds(rbase, ROWS_PER_TILE)
    sl_r = pl.ds(NT * ROWS_PER_TILE, N_NODES - NT * ROWS_PER_TILE)

    @pl.when(c == 0)
    def _():
        pltpu.sync_copy(acc.at[sl], out_lo.at[sl])

        @pl.when(s == NT - 1)
        def _():
            pltpu.sync_copy(acc.at[sl_r], out_lo.at[sl_r])

    @pl.when(c == 1)
    def _():
        pltpu.sync_copy(acc.at[sl], out_hi.at[sl])

        @pl.when(s == NT - 1)
        def _():
            pltpu.sync_copy(acc.at[sl_r], out_hi.at[sl_r])


GB = 10
RB = 1024
EB = N_EDGES // GB


def _tc_body(alo, ahi, em, pred, fm, w1, w2r, out):
    i = pl.program_id(0)

    @pl.when(i == 0)
    def _():
        out[0, 0] = 0.0

    sfm = 1.0 / (1.0 + jnp.exp(-fm[...]))
    hid = lax.dot_general(
        alo[...] * sfm[:, :HALF], w1[:HALF, :],
        (((1,), (0,)), ((), ())),
        precision=lax.Precision.HIGHEST, preferred_element_type=jnp.float32,
    ) + lax.dot_general(
        ahi[...] * sfm[:, HALF:], w1[HALF:, :],
        (((1,), (0,)), ((), ())),
        precision=lax.Precision.HIGHEST, preferred_element_type=jnp.float32,
    )
    hid = jnp.maximum(hid, 0.0)
    lg = lax.dot_general(
        hid, w2r[...], (((1,), (1,)), ((), ())),
        precision=lax.Precision.HIGHEST, preferred_element_type=jnp.float32,
    )[:, 0]
    mse_part = jnp.sum((lg - pred[...].reshape(RB)) ** 2) / N_NODES

    ew = 1.0 / (1.0 + jnp.exp(-em[...]))
    ent_e = -ew * jnp.log(ew + EPS) - (1.0 - ew) * jnp.log(1.0 - ew + EPS)
    contrib = mse_part + ALPHA1 * jnp.sum(ew) + ALPHA2 * jnp.sum(ent_e) / N_EDGES

    out[0, 0] += contrib

    @pl.when(i == GB - 1)
    def _():
        ent_f = -sfm * jnp.log(sfm + EPS) - (1.0 - sfm) * jnp.log(1.0 - sfm + EPS)
        out[0, 0] += BETA1 * jnp.mean(sfm) + BETA2 * jnp.mean(ent_f)


_tc_loss = pl.pallas_call(
    _tc_body,
    grid=(GB,),
    in_specs=[
        pl.BlockSpec((RB, HALF), lambda i: (i, 0)),
        pl.BlockSpec((RB, HALF), lambda i: (i, 0)),
        pl.BlockSpec((1, EB // 128, 128), lambda i: (i, 0, 0)),
        pl.BlockSpec((1, RB // 128, 128), lambda i: (i, 0, 0)),
        pl.BlockSpec((1, D_FEAT), lambda i: (0, 0)),
        pl.BlockSpec((D_FEAT, D_FEAT), lambda i: (0, 0)),
        pl.BlockSpec((1, D_FEAT), lambda i: (0, 0)),
    ],
    out_specs=pl.BlockSpec((1, 1), lambda i: (0, 0), memory_space=pltpu.SMEM),
    out_shape=jax.ShapeDtypeStruct((1, 1), jnp.float32),
)


def kernel(feat, edge_index, feat_mask, edge_mask, W1, W2, pred_value):
    feat2 = feat.reshape(2 * N_NODES, HALF)
    src_h = edge_index[0]
    dst_h = edge_index[1]
    alo, ahi = _make_sc_edge_aggregate()(feat2, src_h, dst_h, edge_mask)
    pad_n = GB * RB - N_NODES
    alo_p = jnp.pad(alo, ((0, pad_n), (0, 0)))
    ahi_p = jnp.pad(ahi, ((0, pad_n), (0, 0)))
    pred_p = jnp.pad(pred_value, (0, pad_n)).reshape(GB, RB // 128, 128)
    em2 = edge_mask.reshape(GB, EB // 128, 128)
    w2r = W2.reshape(1, D_FEAT)
    out = _tc_loss(alo_p, ahi_p, em2, pred_p, feat_mask, W1, w2r)
    return out[0, 0]
